# baseline (speedup 1.0000x reference)
import functools

import jax
import jax.numpy as jnp
from jax import lax
from jax.experimental import pallas as pl
from jax.experimental.pallas import tpu as pltpu
from jax.experimental.pallas import tpu_sc as plsc

EPS = 1e-12
NC = 2
NS = 16
NW = NC * NS
L = 16
PB = 128
PPT = 640
SPLIT = 80
SP2 = 72
GR = 4

_GDN = lax.GatherDimensionNumbers(
    offset_dims=(), collapsed_slice_dims=(0,), start_index_map=(0,)
)


def _permute(v, p):
    return lax.gather(
        v, p[:, None], _GDN, slice_sizes=(1,),
        mode=lax.GatherScatterMode.PROMISE_IN_BOUNDS,
    )


def _xlane_sum(v, perms):
    for p in perms:
        v = v + _permute(v, p)
    return v


def _rsqrt(v):
    vi = lax.bitcast_convert_type(v, jnp.int32)
    yi = jnp.full((L,), 0x5F3759DF, jnp.int32) - lax.shift_right_arithmetic(vi, 1)
    y = lax.bitcast_convert_type(yi, jnp.float32)
    for _ in range(3):
        y = y * (1.5 - 0.5 * v * y * y)
    return y


def _sc_kernel(x_hbm, tt_hbm, sym_hbm, typ_hbm, g_hbm, b_hbm, out_hbm,
               xi, ti, ci, rows, combo, g_v, b_v,
               ix0, ix1, it0, it1, gs0, gs1, gt0, gt1, os0, os1,
               *, rows_per_w, V, T, S, D):
    cid = lax.axis_index("c")
    sid = lax.axis_index("s")
    wid = sid * NC + cid
    pltpu.sync_copy(g_hbm, g_v)
    pltpu.sync_copy(b_hbm, b_v)
    nj = D // L
    gs = tuple(g_v[pl.ds(j * L, L)] for j in range(nj))
    bs = tuple(b_v[pl.ds(j * L, L)] for j in range(nj))
    lane = lax.iota(jnp.int32, L)
    perms = tuple(jnp.bitwise_xor(lane, k) for k in (8, 4, 2, 1))
    ixsems = (ix0, ix1)
    itsems = (it0, it1)
    g1sems = (gs0, gs1)
    g2sems = (gt0, gt1)
    osems = (os0, os1)

    p0 = sid * PPT
    for batch in range(PPT // PB):
        pb = p0 + batch * PB
        for g in range(PB // L):
            pv = lane + (pb + g * L)
            q = pv // T
            ci[0, 0, pl.ds(g * L, L)] = jnp.minimum(q, V - 1)
            ci[0, 1, pl.ds(g * L, L)] = pv - q * T
        cps = pltpu.make_async_copy(
            sym_hbm.at[ci.at[0, 0, pl.ds(0, PB)]], rows.at[0, pl.ds(0, PB)], gs0)
        cpt = pltpu.make_async_copy(
            typ_hbm.at[ci.at[0, 1, pl.ds(0, PB)]], rows.at[1, pl.ds(0, PB)], gt0)
        cps.start()
        cpt.start()
        cps.wait()
        cpt.wait()
        r0, r1 = rows.at[0], rows.at[1]

        @plsc.parallel_loop(0, PB, 1, unroll=4)
        def _pair(i):
            es = []
            for j in range(nj):
                es.append(r0[i, pl.ds(j * L, L)] + r1[i, pl.ds(j * L, L)])
            acc = es[0]
            for j in range(1, nj):
                acc = acc + es[j]
            acc2 = es[0] * es[0]
            for j in range(1, nj):
                acc2 = acc2 + es[j] * es[j]
            mean = _xlane_sum(acc, perms) * (1.0 / D)
            meansq = _xlane_sum(acc2, perms) * (1.0 / D)
            var = meansq - mean * mean
            rstd = _rsqrt(var + EPS)
            mrs = mean * rstd
            for j in range(nj):
                a = gs[j] * rstd
                cc = bs[j] - gs[j] * mrs
                r0[i, pl.ds(j * L, L)] = es[j] * a + cc

        st = pltpu.make_async_copy(
            rows.at[0, pl.ds(0, PB)], combo.at[pl.ds(pb, PB)], os0)
        st.start()
        st.wait()
    plsc.subcore_barrier()

    ngrp = rows_per_w // GR
    grp0 = wid * ngrp
    gtok = GR * S
    nv = (S + L - 1) // L
    shifts = sorted({(r * S) % L for r in range(GR)})
    plo = {sh: (lane + sh) % L for sh in shifts}
    sel = {sh: lane < (L - sh) for sh in shifts}

    def _idxcopies(G, bg):
        f0 = G * gtok
        cpx = pltpu.make_async_copy(
            x_hbm.at[pl.ds(f0, gtok)], xi.at[bg, pl.ds(0, gtok)], ixsems[bg])
        cpt = pltpu.make_async_copy(
            tt_hbm.at[pl.ds(f0, gtok)], ti.at[bg, pl.ds(0, gtok)], itsems[bg])
        return cpx, cpt

    def _outcopy(c, br):
        return pltpu.make_async_copy(
            rows.at[br, pl.ds(0, S)], out_hbm.at[c], osems[br])

    for bg in range(2):
        cpx, cpt = _idxcopies(grp0 + bg, bg)
        cpx.start()
        cpt.start()

    def grp_body(k, carry):
        for bg in range(2):
            gg = 2 * k + bg
            G = grp0 + gg
            cpx, cpt = _idxcopies(G, bg)
            cpx.wait()
            cpt.wait()

            for r in range(GR):
                f = r * S
                sh = f % L
                a0 = f - sh
                for g in range(nv):
                    if sh == 0:
                        xv = xi[bg, pl.ds(a0 + g * L, L)]
                        tv = ti[bg, pl.ds(a0 + g * L, L)]
                    else:
                        xw0 = xi[bg, pl.ds(a0 + g * L, L)]
                        xw1 = xi[bg, pl.ds(a0 + (g + 1) * L, L)]
                        xv = jnp.where(sel[sh], _permute(xw0, plo[sh]),
                                       _permute(xw1, plo[sh]))
                        tw0 = ti[bg, pl.ds(a0 + g * L, L)]
                        tw1 = ti[bg, pl.ds(a0 + (g + 1) * L, L)]
                        tv = jnp.where(sel[sh], _permute(tw0, plo[sh]),
                                       _permute(tw1, plo[sh]))
                    cv = xv * T + tv
                    if (g + 1) * L > S:
                        cv = jnp.minimum(jnp.maximum(cv, 0), NS * PPT - 1)
                    ci[bg, r, pl.ds(g * L, L)] = cv

            @pl.when(gg + 2 < ngrp)
            def _():
                cpx2, cpt2 = _idxcopies(G + 2, bg)
                cpx2.start()
                cpt2.start()

            for r in range(GR):
                c = gg * GR + r
                br = r & 1

                @pl.when(c >= 2)
                def _():
                    _outcopy(wid * rows_per_w + c - 2, br).wait()

                cp1 = pltpu.make_async_copy(
                    combo.at[ci.at[bg, r, pl.ds(0, SPLIT)]],
                    rows.at[br, pl.ds(0, SPLIT)], g1sems[br])
                cp2 = pltpu.make_async_copy(
                    combo.at[ci.at[bg, r, pl.ds(SPLIT, SP2)]],
                    rows.at[br, pl.ds(SPLIT, SP2)], g2sems[br])
                cp1.start()
                cp2.start()
                cp1.wait()
                cp2.wait()
                _outcopy(wid * rows_per_w + c, br).start()
        return carry

    lax.fori_loop(0, ngrp // 2, grp_body, 0)
    for r in range(2):
        c = rows_per_w - 2 + r
        _outcopy(wid * rows_per_w + c, c & 1).wait()


def kernel(x, token_types, symbol_table, token_type_table, gamma, beta):
    B, S = x.shape
    V, D = symbol_table.shape
    T = token_type_table.shape[0]
    N = B * S
    rows_per_w = B // NW
    gtok = GR * S
    nv = (S + L - 1) // L
    pad = gtok + 2 * L
    assert B % (NW * 2 * GR) == 0
    assert gtok % 8 == 0
    assert NS * PPT >= V * T and PPT % PB == 0
    assert SPLIT % 8 == 0 and SP2 % 8 == 0 and SPLIT <= 128 and SP2 <= 128
    assert SPLIT + SP2 >= S and SPLIT + SP2 <= ((S + 7) // 8) * 8 + 8
    assert S >= PB and (GR - 1) * S + nv * L + L <= pad

    xf = x.reshape(N).astype(jnp.int32)
    tf = token_types.reshape(N).astype(jnp.int32)

    mesh = plsc.VectorSubcoreMesh(
        core_axis_name="c", subcore_axis_name="s", num_cores=NC, num_subcores=NS
    )
    run = pl.kernel(
        functools.partial(_sc_kernel, rows_per_w=rows_per_w,
                          V=V, T=T, S=S, D=D),
        out_type=jax.ShapeDtypeStruct((B, S, D), jnp.float32),
        mesh=mesh,
        compiler_params=pltpu.CompilerParams(
            use_tc_tiling_on_sc=False, needs_layout_passes=False
        ),
        scratch_types=[
            pltpu.VMEM((2, pad), jnp.int32),
            pltpu.VMEM((2, pad), jnp.int32),
            pltpu.VMEM((2, GR, ((S + L - 1) // L) * L), jnp.int32),
            pltpu.VMEM((2, SPLIT + SP2, D), jnp.float32),
            pltpu.VMEM_SHARED((NS * PPT, D), jnp.float32),
            pltpu.VMEM((D,), jnp.float32),
            pltpu.VMEM((D,), jnp.float32),
        ] + [pltpu.SemaphoreType.DMA] * 10,
    )
    return run(xf, tf, symbol_table, token_type_table, gamma, beta)


# speedup vs baseline: 2.9777x; 2.9777x over previous
import functools

import jax
import jax.numpy as jnp
from jax import lax
from jax.experimental import pallas as pl
from jax.experimental.pallas import tpu as pltpu
from jax.experimental.pallas import tpu_sc as plsc

EPS = 1e-12
NC = 2
NS = 16
NW = NC * NS
L = 16
PB = 64
PPT = 640
BBLK = 64
NPIECE = 8

_GDN = lax.GatherDimensionNumbers(
    offset_dims=(), collapsed_slice_dims=(0,), start_index_map=(0,)
)


def _permute(v, p):
    return lax.gather(
        v, p[:, None], _GDN, slice_sizes=(1,),
        mode=lax.GatherScatterMode.PROMISE_IN_BOUNDS,
    )


def _xlane_sum(v, perms):
    for p in perms:
        v = v + _permute(v, p)
    return v


def _rsqrt(v):
    vi = lax.bitcast_convert_type(v, jnp.int32)
    yi = jnp.full((L,), 0x5F3759DF, jnp.int32) - lax.shift_right_arithmetic(vi, 1)
    y = lax.bitcast_convert_type(yi, jnp.float32)
    for _ in range(3):
        y = y * (1.5 - 0.5 * v * y * y)
    return y


def _sc_kernel(x_hbm, tt_hbm, sym_hbm, typ_hbm, g_hbm, b_hbm, out_hbm,
               xa, ta, cit, rows, combo, g_v, b_v,
               ix0, ix1, it0, it1, gs0, gs1, os0, os1,
               *, bw, V, T, S, D):
    cid = lax.axis_index("c")
    sid = lax.axis_index("s")
    wid = sid * NC + cid
    pltpu.sync_copy(g_hbm, g_v)
    pltpu.sync_copy(b_hbm, b_v)
    nj = D // L
    gs = tuple(g_v[pl.ds(j * L, L)] for j in range(nj))
    bs = tuple(b_v[pl.ds(j * L, L)] for j in range(nj))
    lane = lax.iota(jnp.int32, L)
    perms = tuple(jnp.bitwise_xor(lane, k) for k in (8, 4, 2, 1))
    ixsems = (ix0, ix1)
    itsems = (it0, it1)
    gsems = (gs0, gs1)
    osems = (os0, os1)

    p0 = sid * PPT
    for batch in range(PPT // PB):
        pb = p0 + batch * PB
        for g in range(PB // L):
            pv = lane + (pb + g * L)
            q = pv // T
            xa[0, pl.ds(g * L, L)] = jnp.minimum(q, V - 1)
            ta[0, pl.ds(g * L, L)] = pv - q * T
        cps = pltpu.make_async_copy(
            sym_hbm.at[xa.at[0, pl.ds(0, PB)]], rows.at[0], gs0)
        cpt = pltpu.make_async_copy(
            typ_hbm.at[ta.at[0, pl.ds(0, PB)]], rows.at[1], gs1)
        cps.start()
        cpt.start()
        cps.wait()
        cpt.wait()
        r0, r1 = rows.at[0], rows.at[1]

        @plsc.parallel_loop(0, PB, 1, unroll=2)
        def _pair(i):
            es = []
            for j in range(nj):
                es.append(r0[i, pl.ds(j * L, L)] + r1[i, pl.ds(j * L, L)])
            acc = es[0]
            for j in range(1, nj):
                acc = acc + es[j]
            acc2 = es[0] * es[0]
            for j in range(1, nj):
                acc2 = acc2 + es[j] * es[j]
            mean = _xlane_sum(acc, perms) * (1.0 / D)
            meansq = _xlane_sum(acc2, perms) * (1.0 / D)
            var = meansq - mean * mean
            rstd = _rsqrt(var + EPS)
            mrs = mean * rstd
            for j in range(nj):
                a = gs[j] * rstd
                cc = bs[j] - gs[j] * mrs
                r0[i, pl.ds(j * L, L)] = es[j] * a + cc

        st = pltpu.make_async_copy(rows.at[0], combo.at[pl.ds(pb, PB)], os0)
        st.start()
        st.wait()
    plsc.subcore_barrier()

    b00 = wid * bw
    f00 = b00 * S
    ntok = bw * S
    plen = ntok // NPIECE

    def _piece(p, bp):
        cpx = pltpu.make_async_copy(
            x_hbm.at[pl.ds(f00 + p * plen, plen)], xa.at[bp], ixsems[bp])
        cpt = pltpu.make_async_copy(
            tt_hbm.at[pl.ds(f00 + p * plen, plen)], ta.at[bp], itsems[bp])
        return cpx, cpt

    for bp in range(2):
        cpx, cpt = _piece(bp, bp)
        cpx.start()
        cpt.start()
    for p in range(NPIECE):
        bp = p % 2
        cpx, cpt = _piece(p, bp)
        cpx.wait()
        cpt.wait()

        @plsc.parallel_loop(0, plen, L)
        def _combine(o):
            fl = (p * plen + o) + lane
            flf = fl.astype(jnp.float32) + 0.5
            bl = (flf * (1.0 / S)).astype(jnp.int32)
            sv = fl - bl * S
            cv = xa[bp, pl.ds(o, L)] * T + ta[bp, pl.ds(o, L)]
            plsc.store_scatter(cit, [sv, bl], cv)

        if p + 2 < NPIECE:
            cpx, cpt = _piece(p + 2, bp)
            cpx.start()
            cpt.start()

    def _gather(s, q):
        return pltpu.make_async_copy(
            combo.at[cit.at[s, pl.ds(q * BBLK, BBLK)]], rows.at[q], gsems[q])

    def _outcopy(s, q):
        dst = out_hbm.at[s, pl.ds(b00 + q * BBLK, BBLK)]
        return pltpu.make_async_copy(rows.at[q], dst, osems[q])

    def s_body(s, carry):
        for q in range(2):
            @pl.when(s >= 1)
            def _():
                _outcopy(s - 1, q).wait()

            cp = _gather(s, q)
            cp.start()
            cp.wait()
            _outcopy(s, q).start()
        return carry

    lax.fori_loop(0, S, s_body, 0)
    for q in range(2):
        _outcopy(S - 1, q).wait()


def kernel(x, token_types, symbol_table, token_type_table, gamma, beta):
    B, S = x.shape
    V, D = symbol_table.shape
    T = token_type_table.shape[0]
    N = B * S
    bw = B // NW
    assert B % NW == 0 and bw == 2 * BBLK
    assert (bw * S) % (NPIECE * L) == 0 and (bw * S // NPIECE) % 8 == 0
    assert NS * PPT >= V * T and PPT % PB == 0

    xf = x.reshape(N).astype(jnp.int32)
    tf = token_types.reshape(N).astype(jnp.int32)

    mesh = plsc.VectorSubcoreMesh(
        core_axis_name="c", subcore_axis_name="s", num_cores=NC, num_subcores=NS
    )
    run = pl.kernel(
        functools.partial(_sc_kernel, bw=bw, V=V, T=T, S=S, D=D),
        out_type=jax.ShapeDtypeStruct((S, B, D), jnp.float32),
        mesh=mesh,
        compiler_params=pltpu.CompilerParams(
            use_tc_tiling_on_sc=False, needs_layout_passes=False
        ),
        scratch_types=[
            pltpu.VMEM((2, B // NW * S // NPIECE), jnp.int32),
            pltpu.VMEM((2, B // NW * S // NPIECE), jnp.int32),
            pltpu.VMEM((S, B // NW), jnp.int32),
            pltpu.VMEM((2, PB, D), jnp.float32),
            pltpu.VMEM_SHARED((NS * PPT, D), jnp.float32),
            pltpu.VMEM((D,), jnp.float32),
            pltpu.VMEM((D,), jnp.float32),
        ] + [pltpu.SemaphoreType.DMA] * 8,
    )
    out = run(xf, tf, symbol_table, token_type_table, gamma, beta)
    return jnp.transpose(out, (1, 0, 2))
